# baseline (speedup 1.0000x reference)
import jax
import jax.numpy as jnp
from jax import lax
from jax.experimental import pallas as pl

B = 64
P = 32768
C = 21
TP = 2048
TP8 = TP // 8
NPT = P // TP
P8 = P // 8
THR = 0.3
F32 = jnp.float32


def _iou_terms(cx, cy, w, h, px, py, pw, ph):
    bx1 = cx - w * 0.5
    by1 = cy - h * 0.5
    bx2 = cx + w * 0.5
    by2 = cy + h * 0.5
    px1 = px - pw * 0.5
    py1 = py - ph * 0.5
    px2 = px + pw * 0.5
    py2 = py + ph * 0.5
    wx = jnp.maximum(jnp.minimum(bx2, px2) - jnp.maximum(bx1, px1), 0.0)
    wy = jnp.maximum(jnp.minimum(by2, py2) - jnp.maximum(by1, py1), 0.0)
    inter = wx * wy
    a1 = w * h
    a2 = pw * ph
    return inter / (a1 + a2 - inter + 1e-10)


def _best_kernel(tgt_ref, pr_ref, best_ref):
    cx = tgt_ref[0, 0, 0]
    cy = tgt_ref[0, 0, 1]
    w = tgt_ref[0, 0, 2]
    h = tgt_ref[0, 0, 3]
    px, py, pw, ph = pr_ref[0], pr_ref[1], pr_ref[2], pr_ref[3]
    iou = _iou_terms(cx, cy, w, h, px, py, pw, ph)
    m = jnp.max(iou)
    r_i = lax.broadcasted_iota(jnp.int32, iou.shape, 0).astype(F32)
    c_i = lax.broadcasted_iota(jnp.int32, iou.shape, 1).astype(F32)
    gp = r_i * float(P8) + c_i
    best = jnp.min(jnp.where(iou == m, gp, F32(P)))
    best_ref[...] = jnp.full((1, 1, 128), best, F32)


def _main_kernel(tgt_ref, best_ref, pr_ref, sc_ref, lc_ref, ce_ref, acc_ref):
    npt = pl.program_id(1)
    cx = tgt_ref[0, 0, 0]
    cy = tgt_ref[0, 0, 1]
    w = tgt_ref[0, 0, 2]
    h = tgt_ref[0, 0, 3]
    sn = tgt_ref[0, 0, 5]
    cs = tgt_ref[0, 0, 6]
    lab = tgt_ref[0, 0, 7]
    best = best_ref[0, 0, 0]
    pt = pr_ref[:, 0]
    px, py, pw, ph = pt[0], pt[1], pt[2], pt[3]

    iou = _iou_terms(cx, cy, w, h, px, py, pw, ph)
    r_i = lax.broadcasted_iota(jnp.int32, iou.shape, 0).astype(F32)
    c_i = lax.broadcasted_iota(jnp.int32, iou.shape, 1).astype(F32)
    gp = lax.convert_element_type(npt * TP, F32) + r_i * float(TP8) + c_i
    pos = (iou >= THR) | (gp == best)
    posf = pos.astype(F32)
    npos_t = jnp.sum(posf)

    s3 = sc_ref[0].reshape(8, TP8, C)
    i21 = (lax.broadcasted_iota(jnp.int32, (C, C), 0)
           == lax.broadcasted_iota(jnp.int32, (C, C), 1)).astype(F32)
    st = lax.dot_general(i21, s3, (((1,), (2,)), ((), ())),
                         preferred_element_type=F32)
    m = jnp.max(st, axis=0)
    e = jnp.exp(st - m[None])
    lse = jnp.log(jnp.sum(e, axis=0)) + m
    oh = (lax.broadcasted_iota(jnp.int32, (1, C), 1).astype(F32) == lab).astype(F32)
    slab = lax.dot_general(oh, st, (((1,), (0,)), ((), ())),
                           preferred_element_type=F32)[0]
    s0 = st[0]
    ce = lse - jnp.where(pos, slab, s0)
    conf_t = jnp.sum(ce * posf)
    ce_ref[0] = jnp.where(pos, 0.0, ce)

    l3 = lc_ref[0].reshape(8, TP8, 6)
    i6 = (lax.broadcasted_iota(jnp.int32, (6, 6), 0)
          == lax.broadcasted_iota(jnp.int32, (6, 6), 1)).astype(F32)
    lt = lax.dot_general(i6, l3, (((1,), (2,)), ((), ())),
                         preferred_element_type=F32)
    gx = (cx - px) / (pw * 0.1)
    gy = (cy - py) / (ph * 0.1)
    gw = 5.0 * jnp.log(w / pw)
    gh = 5.0 * jnp.log(h / ph)
    ones = jnp.ones_like(px)
    tru = jnp.stack([gx, gy, gw, gh, sn * ones, cs * ones])
    d = lt - tru
    ad = jnp.abs(d)
    sl1 = jnp.where(ad < 1.0, 0.5 * d * d, ad - 0.5)
    ridx = lax.broadcasted_iota(jnp.int32, (6, 1, 1), 0)
    pf3 = posf[None]
    loc_t = jnp.sum(jnp.where(ridx < 4, sl1, 0.0) * pf3)
    ang_t = jnp.sum(jnp.where(ridx >= 4, d * d, 0.0) * pf3)

    lane = lax.broadcasted_iota(jnp.int32, (1, 1, 128), 2)
    vec = (jnp.where(lane == 0, npos_t, 0.0)
           + jnp.where(lane == 1, conf_t, 0.0)
           + jnp.where(lane == 2, loc_t, 0.0)
           + jnp.where(lane == 3, ang_t, 0.0))

    @pl.when(npt == 0)
    def _():
        acc_ref[...] = vec

    @pl.when(npt != 0)
    def _():
        acc_ref[...] = acc_ref[...] + vec


def _mine_kernel(ce_ref, acc_ref, tot_ref):
    b = pl.program_id(0)
    x = ce_ref[0]
    bits = lax.bitcast_convert_type(x, jnp.int32)
    npos = acc_ref[0, 0, 0]
    kf = jnp.minimum(npos * 3.0, F32(P))

    def body(_, lohi):
        lo, hi = lohi
        mid = lo + lax.div(hi - lo, jnp.int32(2))
        cnt = jnp.sum((bits > mid).astype(F32))
        take = cnt >= kf
        return (jnp.where(take, mid, lo), jnp.where(take, hi, mid))

    _, hi = lax.fori_loop(0, 31, body, (jnp.int32(-1), jnp.int32(0x7F800000)))
    vkf = lax.bitcast_convert_type(hi, F32)
    gt = bits > hi
    sum_gt = jnp.sum(jnp.where(gt, x, 0.0))
    cnt_gt = jnp.sum(gt.astype(F32))
    topk = sum_gt + (kf - cnt_gt) * vkf

    lane = lax.broadcasted_iota(jnp.int32, (1, 128), 1)
    vec = (jnp.where(lane == 0, npos, 0.0)
           + jnp.where(lane == 1, acc_ref[0, 0, 1] + topk, 0.0)
           + jnp.where(lane == 2, acc_ref[0, 0, 2], 0.0)
           + jnp.where(lane == 3, acc_ref[0, 0, 3], 0.0))

    @pl.when(b == 0)
    def _():
        tot_ref[...] = vec

    @pl.when(b != 0)
    def _():
        tot_ref[...] = tot_ref[...] + vec


def kernel(predicted_locs, predicted_scores, target, priors_cxcy):
    prt = priors_cxcy.T
    priors_b = prt.reshape(4, 8, P8)
    priors_4 = prt.reshape(4, NPT, 8, TP8)

    best = pl.pallas_call(
        _best_kernel,
        grid=(B,),
        in_specs=[
            pl.BlockSpec((1, 1, 8), lambda b: (b, 0, 0)),
            pl.BlockSpec((4, 8, P8), lambda b: (0, 0, 0)),
        ],
        out_specs=pl.BlockSpec((1, 1, 128), lambda b: (b, 0, 0)),
        out_shape=jax.ShapeDtypeStruct((B, 1, 128), F32),
    )(target, priors_b)

    ce_neg, acc = pl.pallas_call(
        _main_kernel,
        grid=(B, NPT),
        in_specs=[
            pl.BlockSpec((1, 1, 8), lambda b, n: (b, 0, 0)),
            pl.BlockSpec((1, 1, 128), lambda b, n: (b, 0, 0)),
            pl.BlockSpec((4, 1, 8, TP8), lambda b, n: (0, n, 0, 0)),
            pl.BlockSpec((1, TP, C), lambda b, n: (b, n, 0)),
            pl.BlockSpec((1, TP, 6), lambda b, n: (b, n, 0)),
        ],
        out_specs=[
            pl.BlockSpec((1, 8, TP8), lambda b, n: (b, 0, n)),
            pl.BlockSpec((1, 1, 128), lambda b, n: (b, 0, 0)),
        ],
        out_shape=[
            jax.ShapeDtypeStruct((B, 8, P8), F32),
            jax.ShapeDtypeStruct((B, 1, 128), F32),
        ],
    )(target, best, priors_4, predicted_scores, predicted_locs)

    tot = pl.pallas_call(
        _mine_kernel,
        grid=(B,),
        in_specs=[
            pl.BlockSpec((1, 8, P8), lambda b: (b, 0, 0)),
            pl.BlockSpec((1, 1, 128), lambda b: (b, 0, 0)),
        ],
        out_specs=pl.BlockSpec((1, 128), lambda b: (0, 0)),
        out_shape=jax.ShapeDtypeStruct((1, 128), F32),
    )(ce_neg, acc)

    n = tot[0, 0]
    conf = tot[0, 1] / n
    loc = tot[0, 2] / (n * 4.0)
    ang = 25.0 * tot[0, 3] / (n * 2.0)
    return (conf, loc, ang, conf + loc + ang)


# speedup vs baseline: 1.4211x; 1.4211x over previous
import numpy as np
import jax
import jax.numpy as jnp
from jax import lax
from jax.experimental import pallas as pl
from jax.experimental.pallas import tpu as pltpu

B = 64
P = 32768
C = 21
IB = 8
NT = 8
GT = 32
P8 = P // 8
THR = 0.3
F32 = jnp.float32

_s = np.arange(C)[:, None, None]
_l = np.arange(128)[None, :, None]
_p = np.arange(128)[None, None, :]
_f = 128 * _s + _l
W_SSE = (_f // C == _p).astype(np.float32)
W_S0 = (_f == C * _p).astype(np.float32)
D_IDX = (_f - C * _p).astype(np.int32)
_s6 = np.arange(6)[:, None, None]
_f6 = 128 * _s6 + _l
_j6 = np.arange(6 * 128)[None, None, :] // 128
_p6 = np.arange(6 * 128)[None, None, :] % 128
W_LOC = (_f6 == 6 * _p6 + _j6).astype(np.float32)

_BN = (((2,), (1,)), ((1,), (0,)))


def _iou_terms(cx, cy, w, h, px, py, pw, ph):
    bx1 = cx - w * 0.5
    by1 = cy - h * 0.5
    bx2 = cx + w * 0.5
    by2 = cy + h * 0.5
    px1 = px - pw * 0.5
    py1 = py - ph * 0.5
    px2 = px + pw * 0.5
    py2 = py + ph * 0.5
    wx = jnp.maximum(jnp.minimum(bx2, px2) - jnp.maximum(bx1, px1), 0.0)
    wy = jnp.maximum(jnp.minimum(by2, py2) - jnp.maximum(by1, py1), 0.0)
    inter = wx * wy
    return inter / (w * h + pw * ph - inter + 1e-10)


def _best_kernel(tgt_ref, pr_ref, best_ref):
    cx = tgt_ref[0, 0, 0]
    cy = tgt_ref[0, 0, 1]
    w = tgt_ref[0, 0, 2]
    h = tgt_ref[0, 0, 3]
    px, py, pw, ph = pr_ref[0], pr_ref[1], pr_ref[2], pr_ref[3]
    iou = _iou_terms(cx, cy, w, h, px, py, pw, ph)
    m = jnp.max(iou)
    r_i = lax.broadcasted_iota(jnp.int32, iou.shape, 0).astype(F32)
    c_i = lax.broadcasted_iota(jnp.int32, iou.shape, 1).astype(F32)
    gp = r_i * float(P8) + c_i
    best = jnp.min(jnp.where(iou == m, gp, F32(P)))
    best_ref[...] = jnp.full((1, 1, 128), best, F32)


def _main_kernel(tgt_ref, best_ref, pr_ref, sc_ref, lc_ref, wsse_ref,
                 ws0_ref, didx_ref, wloc_ref, acc_ref, tot_ref,
                 wslab_ref, ce_ref):
    n = pl.program_id(1)
    b = pl.program_id(0)
    cx = tgt_ref[0, 0, 0]
    cy = tgt_ref[0, 0, 1]
    w = tgt_ref[0, 0, 2]
    h = tgt_ref[0, 0, 3]
    sn = tgt_ref[0, 0, 5]
    cs = tgt_ref[0, 0, 6]
    lab_i = lax.convert_element_type(tgt_ref[0, 0, 7], jnp.int32)
    best = best_ref[0, 0, 0]
    px, py, pw, ph = (pr_ref[0, 0], pr_ref[1, 0], pr_ref[2, 0], pr_ref[3, 0])

    @pl.when(n == 0)
    def _():
        wslab_ref[...] = (didx_ref[...] == lab_i).astype(F32)

    iou = _iou_terms(cx, cy, w, h, px, py, pw, ph)
    g_i = lax.broadcasted_iota(jnp.int32, iou.shape, 0)
    l_i = lax.broadcasted_iota(jnp.int32, iou.shape, 1)
    gp = ((n * (GT * 128) + g_i * 128 + l_i)).astype(F32)
    pos = (iou >= THR) | (gp == best)
    posf = pos.astype(F32)
    npos_t = jnp.sum(posf)

    x = sc_ref[0]
    e = jnp.exp(x)
    sse = jnp.sum(lax.dot_general(e, wsse_ref[...], _BN,
                                  preferred_element_type=F32), axis=0)
    s0 = jnp.sum(lax.dot_general(x, ws0_ref[...], _BN,
                                 preferred_element_type=F32), axis=0)
    slab = jnp.sum(lax.dot_general(x, wslab_ref[...], _BN,
                                   preferred_element_type=F32), axis=0)
    lse = jnp.log(sse)
    ce0 = lse - s0
    conf_t = jnp.sum(posf * (lse - slab))
    ce_ref[pl.ds(n * GT, GT), :] = jnp.where(pos, 0.0, ce0)

    xl = lc_ref[0]
    lt = jnp.sum(lax.dot_general(xl, wloc_ref[...], _BN,
                                 preferred_element_type=F32), axis=0)
    d0 = lt[:, 0:128] - (cx - px) / (pw * 0.1)
    d1 = lt[:, 128:256] - (cy - py) / (ph * 0.1)
    d2 = lt[:, 256:384] - 5.0 * jnp.log(w / pw)
    d3 = lt[:, 384:512] - 5.0 * jnp.log(h / ph)
    d4 = lt[:, 512:640] - sn
    d5 = lt[:, 640:768] - cs
    loc_t = 0.0
    for d in (d0, d1, d2, d3):
        ad = jnp.abs(d)
        loc_t += jnp.sum(posf * jnp.where(ad < 1.0, 0.5 * d * d, ad - 0.5))
    ang_t = jnp.sum(posf * (d4 * d4 + d5 * d5))

    lane = lax.broadcasted_iota(jnp.int32, (1, 1, 128), 2)
    vec = (jnp.where(lane == 0, npos_t, 0.0)
           + jnp.where(lane == 1, conf_t, 0.0)
           + jnp.where(lane == 2, loc_t, 0.0)
           + jnp.where(lane == 3, ang_t, 0.0))

    @pl.when(n == 0)
    def _():
        acc_ref[...] = vec

    @pl.when(n != 0)
    def _():
        acc_ref[...] = acc_ref[...] + vec

    @pl.when((b == 0) & (n == 0))
    def _():
        tot_ref[...] = jnp.zeros((1, 1, 128), F32)

    @pl.when(n == NT - 1)
    def _():
        x = ce_ref[...]
        bits = lax.bitcast_convert_type(x, jnp.int32)
        npos = acc_ref[0, 0, 0]
        kf = jnp.minimum(npos * 3.0, F32(P))

        def body(_, lohi):
            lo, hi = lohi
            mid = lo + lax.div(hi - lo, jnp.int32(2))
            cnt = jnp.sum((bits > mid).astype(F32))
            take = cnt >= kf
            return (jnp.where(take, mid, lo), jnp.where(take, hi, mid))

        _, hi = lax.fori_loop(0, 31, body,
                              (jnp.int32(-1), jnp.int32(0x7F800000)))
        vkf = lax.bitcast_convert_type(hi, F32)
        gtm = bits > hi
        sum_gt = jnp.sum(jnp.where(gtm, x, 0.0))
        cnt_gt = jnp.sum(gtm.astype(F32))
        topk = sum_gt + (kf - cnt_gt) * vkf
        vec2 = (jnp.where(lane == 0, npos, 0.0)
                + jnp.where(lane == 1, acc_ref[0, 0, 1] + topk, 0.0)
                + jnp.where(lane == 2, acc_ref[0, 0, 2], 0.0)
                + jnp.where(lane == 3, acc_ref[0, 0, 3], 0.0))
        tot_ref[...] = tot_ref[...] + vec2


def kernel(predicted_locs, predicted_scores, target, priors_cxcy):
    prt = priors_cxcy.T
    priors_b = prt.reshape(4, 8, P8)
    priors_m = prt.reshape(4, NT, GT, 128)
    scf = predicted_scores.reshape(B, NT * GT, C, 128)
    lcf = predicted_locs.reshape(B, NT * GT, 6, 128)
    wsse = jnp.asarray(W_SSE)
    ws0 = jnp.asarray(W_S0)
    didx = jnp.asarray(D_IDX)
    wloc = jnp.asarray(W_LOC)

    best = pl.pallas_call(
        _best_kernel,
        grid=(B,),
        in_specs=[
            pl.BlockSpec((1, 1, 8), lambda b: (b, 0, 0)),
            pl.BlockSpec((4, 8, P8), lambda b: (0, 0, 0)),
        ],
        out_specs=pl.BlockSpec((1, 1, 128), lambda b: (b, 0, 0)),
        out_shape=jax.ShapeDtypeStruct((B, 1, 128), F32),
    )(target, priors_b)

    acc, tot3 = pl.pallas_call(
        _main_kernel,
        grid=(B, NT),
        in_specs=[
            pl.BlockSpec((1, 1, 8), lambda b, n: (b, 0, 0)),
            pl.BlockSpec((1, 1, 128), lambda b, n: (b, 0, 0)),
            pl.BlockSpec((4, 1, GT, 128), lambda b, n: (0, n, 0, 0)),
            pl.BlockSpec((1, GT, C, 128), lambda b, n: (b, n, 0, 0)),
            pl.BlockSpec((1, GT, 6, 128), lambda b, n: (b, n, 0, 0)),
            pl.BlockSpec((C, 128, 128), lambda b, n: (0, 0, 0)),
            pl.BlockSpec((C, 128, 128), lambda b, n: (0, 0, 0)),
            pl.BlockSpec((C, 128, 128), lambda b, n: (0, 0, 0)),
            pl.BlockSpec((6, 128, 768), lambda b, n: (0, 0, 0)),
        ],
        out_specs=[
            pl.BlockSpec((1, 1, 128), lambda b, n: (b, 0, 0)),
            pl.BlockSpec((1, 1, 128), lambda b, n: (0, 0, 0)),
        ],
        out_shape=[
            jax.ShapeDtypeStruct((B, 1, 128), F32),
            jax.ShapeDtypeStruct((1, 1, 128), F32),
        ],
        scratch_shapes=[pltpu.VMEM((C, 128, 128), F32),
                        pltpu.VMEM((NT * GT, 128), F32)],
    )(target, best, priors_m, scf, lcf, wsse, ws0, didx, wloc)

    n = tot3[0, 0, 0]
    conf = tot3[0, 0, 1] / n
    loc = tot3[0, 0, 2] / (n * 4.0)
    ang = 25.0 * tot3[0, 0, 3] / (n * 2.0)
    return (conf, loc, ang, conf + loc + ang)


# speedup vs baseline: 1.7942x; 1.2626x over previous
import numpy as np
import jax
import jax.numpy as jnp
from jax import lax
from jax.experimental import pallas as pl
from jax.experimental.pallas import tpu as pltpu

B = 64
P = 32768
C = 21
IB = 8
NT = 1
GT = 256
P8 = P // 8
THR = 0.3
F32 = jnp.float32

_s = np.arange(C)[:, None, None]
_l = np.arange(128)[None, :, None]
_p = np.arange(128)[None, None, :]
_f = 128 * _s + _l
W_SSE = (_f // C == _p).astype(np.float32)
W_S0 = (_f == C * _p).astype(np.float32)
D_IDX = (_f - C * _p).astype(np.int32)
_s6 = np.arange(6)[:, None, None]
_f6 = 128 * _s6 + _l
_j6 = np.arange(6 * 128)[None, None, :] // 128
_p6 = np.arange(6 * 128)[None, None, :] % 128
W_LOC = (_f6 == 6 * _p6 + _j6).astype(np.float32)

_BN = (((2,), (1,)), ((1,), (0,)))


def _iou_terms(cx, cy, w, h, px, py, pw, ph):
    bx1 = cx - w * 0.5
    by1 = cy - h * 0.5
    bx2 = cx + w * 0.5
    by2 = cy + h * 0.5
    px1 = px - pw * 0.5
    py1 = py - ph * 0.5
    px2 = px + pw * 0.5
    py2 = py + ph * 0.5
    wx = jnp.maximum(jnp.minimum(bx2, px2) - jnp.maximum(bx1, px1), 0.0)
    wy = jnp.maximum(jnp.minimum(by2, py2) - jnp.maximum(by1, py1), 0.0)
    inter = wx * wy
    return inter / (w * h + pw * ph - inter + 1e-10)


def _best_kernel(tgt_ref, pr_ref, best_ref):
    cx = tgt_ref[0, 0, 0]
    cy = tgt_ref[0, 0, 1]
    w = tgt_ref[0, 0, 2]
    h = tgt_ref[0, 0, 3]
    px, py, pw, ph = pr_ref[0], pr_ref[1], pr_ref[2], pr_ref[3]
    iou = _iou_terms(cx, cy, w, h, px, py, pw, ph)
    m = jnp.max(iou)
    r_i = lax.broadcasted_iota(jnp.int32, iou.shape, 0).astype(F32)
    c_i = lax.broadcasted_iota(jnp.int32, iou.shape, 1).astype(F32)
    gp = r_i * float(P8) + c_i
    best = jnp.min(jnp.where(iou == m, gp, F32(P)))
    best_ref[...] = jnp.full((1, 1, 128), best, F32)


def _main_kernel(tgt_ref, best_ref, pr_ref, sc_ref, lc_ref, wsse_ref,
                 ws0_ref, didx_ref, wloc_ref, acc_ref, tot_ref,
                 wslab_ref, ce_ref):
    n = pl.program_id(1)
    b = pl.program_id(0)
    cx = tgt_ref[0, 0, 0]
    cy = tgt_ref[0, 0, 1]
    w = tgt_ref[0, 0, 2]
    h = tgt_ref[0, 0, 3]
    sn = tgt_ref[0, 0, 5]
    cs = tgt_ref[0, 0, 6]
    lab_i = lax.convert_element_type(tgt_ref[0, 0, 7], jnp.int32)
    best = best_ref[0, 0, 0]
    px, py, pw, ph = (pr_ref[0, 0], pr_ref[1, 0], pr_ref[2, 0], pr_ref[3, 0])

    @pl.when(n == 0)
    def _():
        wslab_ref[...] = (didx_ref[...] == lab_i).astype(F32)

    iou = _iou_terms(cx, cy, w, h, px, py, pw, ph)
    g_i = lax.broadcasted_iota(jnp.int32, iou.shape, 0)
    l_i = lax.broadcasted_iota(jnp.int32, iou.shape, 1)
    gp = ((n * (GT * 128) + g_i * 128 + l_i)).astype(F32)
    pos = (iou >= THR) | (gp == best)
    posf = pos.astype(F32)
    npos_t = jnp.sum(posf)

    x = sc_ref[0]
    e = jnp.exp(x)
    sse = jnp.sum(lax.dot_general(e, wsse_ref[...], _BN,
                                  preferred_element_type=F32), axis=0)
    s0 = jnp.sum(lax.dot_general(x, ws0_ref[...], _BN,
                                 preferred_element_type=F32), axis=0)
    slab = jnp.sum(lax.dot_general(x, wslab_ref[...], _BN,
                                   preferred_element_type=F32), axis=0)
    lse = jnp.log(sse)
    ce0 = lse - s0
    conf_t = jnp.sum(posf * (lse - slab))
    ce_ref[pl.ds(n * GT, GT), :] = jnp.where(pos, 0.0, ce0)

    xl = lc_ref[0]
    lt = jnp.sum(lax.dot_general(xl, wloc_ref[...], _BN,
                                 preferred_element_type=F32), axis=0)
    d0 = lt[:, 0:128] - (cx - px) / (pw * 0.1)
    d1 = lt[:, 128:256] - (cy - py) / (ph * 0.1)
    d2 = lt[:, 256:384] - 5.0 * jnp.log(w / pw)
    d3 = lt[:, 384:512] - 5.0 * jnp.log(h / ph)
    d4 = lt[:, 512:640] - sn
    d5 = lt[:, 640:768] - cs
    loc_t = 0.0
    for d in (d0, d1, d2, d3):
        ad = jnp.abs(d)
        loc_t += jnp.sum(posf * jnp.where(ad < 1.0, 0.5 * d * d, ad - 0.5))
    ang_t = jnp.sum(posf * (d4 * d4 + d5 * d5))

    lane = lax.broadcasted_iota(jnp.int32, (1, 1, 128), 2)
    vec = (jnp.where(lane == 0, npos_t, 0.0)
           + jnp.where(lane == 1, conf_t, 0.0)
           + jnp.where(lane == 2, loc_t, 0.0)
           + jnp.where(lane == 3, ang_t, 0.0))

    @pl.when(n == 0)
    def _():
        acc_ref[...] = vec

    @pl.when(n != 0)
    def _():
        acc_ref[...] = acc_ref[...] + vec

    @pl.when((b == 0) & (n == 0))
    def _():
        tot_ref[...] = jnp.zeros((1, 1, 128), F32)

    @pl.when(n == NT - 1)
    def _():
        x = ce_ref[...]
        bits = lax.bitcast_convert_type(x, jnp.int32)
        npos = acc_ref[0, 0, 0]
        kf = jnp.minimum(npos * 3.0, F32(P))

        def body(_, lohi):
            lo, hi = lohi
            mid = lo + lax.div(hi - lo, jnp.int32(2))
            cnt = jnp.sum((bits > mid).astype(F32))
            take = cnt >= kf
            return (jnp.where(take, mid, lo), jnp.where(take, hi, mid))

        _, hi = lax.fori_loop(0, 31, body,
                              (jnp.int32(-1), jnp.int32(0x7F800000)))
        vkf = lax.bitcast_convert_type(hi, F32)
        gtm = bits > hi
        sum_gt = jnp.sum(jnp.where(gtm, x, 0.0))
        cnt_gt = jnp.sum(gtm.astype(F32))
        topk = sum_gt + (kf - cnt_gt) * vkf
        vec2 = (jnp.where(lane == 0, npos, 0.0)
                + jnp.where(lane == 1, acc_ref[0, 0, 1] + topk, 0.0)
                + jnp.where(lane == 2, acc_ref[0, 0, 2], 0.0)
                + jnp.where(lane == 3, acc_ref[0, 0, 3], 0.0))
        tot_ref[...] = tot_ref[...] + vec2


def kernel(predicted_locs, predicted_scores, target, priors_cxcy):
    prt = priors_cxcy.T
    priors_b = prt.reshape(4, 8, P8)
    priors_m = prt.reshape(4, NT, GT, 128)
    scf = predicted_scores.reshape(B, NT * GT, C, 128)
    lcf = predicted_locs.reshape(B, NT * GT, 6, 128)
    wsse = jnp.asarray(W_SSE)
    ws0 = jnp.asarray(W_S0)
    didx = jnp.asarray(D_IDX)
    wloc = jnp.asarray(W_LOC)

    best = pl.pallas_call(
        _best_kernel,
        grid=(B,),
        in_specs=[
            pl.BlockSpec((1, 1, 8), lambda b: (b, 0, 0)),
            pl.BlockSpec((4, 8, P8), lambda b: (0, 0, 0)),
        ],
        out_specs=pl.BlockSpec((1, 1, 128), lambda b: (b, 0, 0)),
        out_shape=jax.ShapeDtypeStruct((B, 1, 128), F32),
    )(target, priors_b)

    acc, tot3 = pl.pallas_call(
        _main_kernel,
        grid=(B, NT),
        in_specs=[
            pl.BlockSpec((1, 1, 8), lambda b, n: (b, 0, 0)),
            pl.BlockSpec((1, 1, 128), lambda b, n: (b, 0, 0)),
            pl.BlockSpec((4, 1, GT, 128), lambda b, n: (0, n, 0, 0)),
            pl.BlockSpec((1, GT, C, 128), lambda b, n: (b, n, 0, 0)),
            pl.BlockSpec((1, GT, 6, 128), lambda b, n: (b, n, 0, 0)),
            pl.BlockSpec((C, 128, 128), lambda b, n: (0, 0, 0)),
            pl.BlockSpec((C, 128, 128), lambda b, n: (0, 0, 0)),
            pl.BlockSpec((C, 128, 128), lambda b, n: (0, 0, 0)),
            pl.BlockSpec((6, 128, 768), lambda b, n: (0, 0, 0)),
        ],
        out_specs=[
            pl.BlockSpec((1, 1, 128), lambda b, n: (b, 0, 0)),
            pl.BlockSpec((1, 1, 128), lambda b, n: (0, 0, 0)),
        ],
        out_shape=[
            jax.ShapeDtypeStruct((B, 1, 128), F32),
            jax.ShapeDtypeStruct((1, 1, 128), F32),
        ],
        scratch_shapes=[pltpu.VMEM((C, 128, 128), F32),
                        pltpu.VMEM((NT * GT, 128), F32)],
    )(target, best, priors_m, scf, lcf, wsse, ws0, didx, wloc)

    n = tot3[0, 0, 0]
    conf = tot3[0, 0, 1] / n
    loc = tot3[0, 0, 2] / (n * 4.0)
    ang = 25.0 * tot3[0, 0, 3] / (n * 2.0)
    return (conf, loc, ang, conf + loc + ang)
